# Initial kernel scaffold; baseline (speedup 1.0000x reference)
import jax
import jax.numpy as jnp
from jax.experimental import pallas as pl


def kernel(x, edge_index, W1, b1, W2, b2):
    raise NotImplementedError("write your pallas kernel here")


# speedup vs baseline: 20.0563x; 20.0563x over previous
import functools

import jax
import jax.numpy as jnp
from jax import lax
from jax.experimental import pallas as pl
from jax.experimental.pallas import tpu as pltpu
from jax.experimental.pallas import tpu_sc as plsc

N_NODES = 10000
NPAD = 10240
IN_DIM = 128
HID_DIM = 128
OUT_DIM = 64
N_EDGES = 320000

NC = 2
NS = 16
NW = NC * NS
EPW = N_EDGES // NW
K = 80
CH = EPW // K
RPT = NPAD // NS

_SELU_ALPHA = 1.6732632423543772
_SELU_SCALE = 1.0507009873554805


def _mesh():
    return plsc.VectorSubcoreMesh(core_axis_name="c", subcore_axis_name="s")


def _sc_degree(dst_r):
    @functools.partial(
        pl.kernel,
        out_type=jax.ShapeDtypeStruct((NC, NPAD), jnp.float32),
        mesh=_mesh(),
        scratch_types=[
            pltpu.VMEM((CH, K), jnp.int32),
            pltpu.VMEM((K,), jnp.float32),
            pltpu.VMEM((RPT,), jnp.float32),
            pltpu.VMEM_SHARED((NPAD,), jnp.float32),
        ],
    )
    def deg_kernel(dst_hbm, out_hbm, dstv, onesv, zv, acc_sh):
        c = lax.axis_index("c")
        s = lax.axis_index("s")
        wid = s * NC + c

        def fill(i, _):
            zv[pl.ds(i * 16, 16)] = jnp.zeros((16,), jnp.float32)
            return 0

        lax.fori_loop(0, RPT // 16, fill, 0)
        for i in range(K // 16):
            onesv[pl.ds(i * 16, 16)] = jnp.ones((16,), jnp.float32)
        pltpu.sync_copy(zv, acc_sh.at[pl.ds(s * RPT, RPT)])
        pltpu.sync_copy(dst_hbm.at[wid], dstv)
        plsc.subcore_barrier()

        def body(j, _):
            pltpu.sync_copy(onesv, acc_sh.at[dstv.at[j]], add=True)
            return 0

        lax.fori_loop(0, CH, body, 0)
        plsc.subcore_barrier()
        pltpu.sync_copy(acc_sh.at[pl.ds(s * RPT, RPT)],
                        out_hbm.at[c, pl.ds(s * RPT, RPT)])

    return deg_kernel(dst_r)


def _sc_scatter(hs, src_r, dst_r, d):
    @functools.partial(
        pl.kernel,
        out_type=jax.ShapeDtypeStruct((NC, NPAD, d), jnp.float32),
        mesh=_mesh(),
        scratch_types=[
            pltpu.VMEM((CH, K), jnp.int32),
            pltpu.VMEM((CH, K), jnp.int32),
            pltpu.VMEM((K, d), jnp.float32),
            pltpu.VMEM_SHARED((NPAD, d), jnp.float32),
            pltpu.SemaphoreType.DMA,
        ],
    )
    def scat_kernel(hs_hbm, src_hbm, dst_hbm, out_hbm,
                    srcv, dstv, rows, acc_sh, sem):
        c = lax.axis_index("c")
        s = lax.axis_index("s")
        wid = s * NC + c

        def zfill(r, _):
            for i in range(d // 16):
                rows[r, pl.ds(i * 16, 16)] = jnp.zeros((16,), jnp.float32)
            return 0

        lax.fori_loop(0, K, zfill, 0)
        for i in range(RPT // K):
            pltpu.sync_copy(rows, acc_sh.at[pl.ds(s * RPT + i * K, K)])
        pltpu.sync_copy(src_hbm.at[wid], srcv)
        pltpu.sync_copy(dst_hbm.at[wid], dstv)
        plsc.subcore_barrier()

        def body(j, _):
            pltpu.async_copy(hs_hbm.at[srcv.at[j]], rows, sem).wait()
            pltpu.sync_copy(rows, acc_sh.at[dstv.at[j]], add=True)
            return 0

        lax.fori_loop(0, CH, body, 0)
        plsc.subcore_barrier()
        pltpu.sync_copy(acc_sh.at[pl.ds(s * RPT, RPT)],
                        out_hbm.at[c, pl.ds(s * RPT, RPT)])

    return scat_kernel(hs, src_r, dst_r)


def _tc1_body(x_ref, w_ref, degt_ref, hs_ref, dinv_ref):
    deg = degt_ref[:N_NODES, 0:1] + degt_ref[:N_NODES, 1:2] + 1.0
    dinv = lax.rsqrt(deg)
    h = jnp.dot(x_ref[...], w_ref[...], preferred_element_type=jnp.float32)
    hs_ref[...] = dinv * h
    dinv_ref[...] = dinv


def _tc2_body(p_ref, hs1_ref, dinv_ref, b1_ref, w2_ref, hs2_ref):
    dinv = dinv_ref[...]
    z = dinv * (p_ref[0, :N_NODES, :] + p_ref[1, :N_NODES, :] + hs1_ref[...])
    z = z + b1_ref[...]
    a = _SELU_SCALE * jnp.where(z > 0, z, _SELU_ALPHA * (jnp.exp(z) - 1.0))
    h2 = jnp.dot(a, w2_ref[...], preferred_element_type=jnp.float32)
    hs2_ref[:, :OUT_DIM] = dinv * h2
    hs2_ref[:, OUT_DIM:] = jnp.zeros((N_NODES, HID_DIM - OUT_DIM), jnp.float32)


def _tc3_body(q_ref, hs2_ref, dinv_ref, b2_ref, out_ref):
    z = dinv_ref[...] * (q_ref[0, :N_NODES, :OUT_DIM]
                         + q_ref[1, :N_NODES, :OUT_DIM]
                         + hs2_ref[:, :OUT_DIM])
    out_ref[...] = z + b2_ref[...]


def kernel(x, edge_index, W1, b1, W2, b2):
    ei = edge_index.astype(jnp.int32)
    src_r = ei[0].reshape(NW, CH, K)
    dst_r = ei[1].reshape(NW, CH, K)

    deg_p = _sc_degree(dst_r)
    degt = deg_p.T

    hs1, dinv = pl.pallas_call(
        _tc1_body,
        out_shape=(jax.ShapeDtypeStruct((N_NODES, HID_DIM), jnp.float32),
                   jax.ShapeDtypeStruct((N_NODES, 1), jnp.float32)),
    )(x, W1, degt)

    p = _sc_scatter(hs1, src_r, dst_r, HID_DIM)

    hs2 = pl.pallas_call(
        _tc2_body,
        out_shape=jax.ShapeDtypeStruct((N_NODES, HID_DIM), jnp.float32),
    )(p, hs1, dinv, b1.reshape(1, HID_DIM), W2)

    q = _sc_scatter(hs2, src_r, dst_r, HID_DIM)

    out = pl.pallas_call(
        _tc3_body,
        out_shape=jax.ShapeDtypeStruct((N_NODES, OUT_DIM), jnp.float32),
    )(q, hs2, dinv, b2.reshape(1, OUT_DIM))
    return out


# speedup vs baseline: 23.6692x; 1.1801x over previous
import functools

import jax
import jax.numpy as jnp
from jax import lax
from jax.experimental import pallas as pl
from jax.experimental.pallas import tpu as pltpu
from jax.experimental.pallas import tpu_sc as plsc

N_NODES = 10000
NPAD = 10240
IN_DIM = 128
HID_DIM = 128
OUT_DIM = 64
N_EDGES = 320000

NC = 2
NS = 16
NW = NC * NS
EPW = 10240
E_PAD = NW * EPW
K = 64
CH = EPW // K
IB = 8
NB = CH // IB
RPT = NPAD // NS

_SELU_ALPHA = 1.6732632423543772
_SELU_SCALE = 1.0507009873554805


def _mesh():
    return plsc.VectorSubcoreMesh(core_axis_name="c", subcore_axis_name="s")


def _sc_degree(dst_r):
    @functools.partial(
        pl.kernel,
        out_type=jax.ShapeDtypeStruct((NC, NPAD), jnp.float32),
        mesh=_mesh(),
        scratch_types=[
            pltpu.VMEM((CH, K), jnp.int32),
            pltpu.VMEM((K,), jnp.float32),
            pltpu.VMEM((RPT,), jnp.float32),
            pltpu.VMEM_SHARED((NPAD,), jnp.float32),
        ],
    )
    def deg_kernel(dst_hbm, out_hbm, dstv, onesv, zv, acc_sh):
        c = lax.axis_index("c")
        s = lax.axis_index("s")
        wid = s * NC + c

        def fill(i, _):
            zv[pl.ds(i * 16, 16)] = jnp.zeros((16,), jnp.float32)
            return 0

        lax.fori_loop(0, RPT // 16, fill, 0)
        for i in range(K // 16):
            onesv[pl.ds(i * 16, 16)] = jnp.ones((16,), jnp.float32)
        pltpu.sync_copy(zv, acc_sh.at[pl.ds(s * RPT, RPT)])
        pltpu.sync_copy(dst_hbm.at[wid], dstv)
        plsc.subcore_barrier()

        def body(j, _):
            pltpu.sync_copy(onesv, acc_sh.at[dstv.at[j]], add=True)
            return 0

        lax.fori_loop(0, CH, body, 0)
        plsc.subcore_barrier()
        pltpu.sync_copy(acc_sh.at[pl.ds(s * RPT, RPT)],
                        out_hbm.at[c, pl.ds(s * RPT, RPT)])

    return deg_kernel(dst_r)


def _sc_scatter(hs, e_r, d, tc_tiling=True):
    @functools.partial(
        pl.kernel,
        out_type=jax.ShapeDtypeStruct((NC, NPAD, d), jnp.float32),
        mesh=_mesh(),
        compiler_params=pltpu.CompilerParams(use_tc_tiling_on_sc=tc_tiling),
        scratch_types=[
            pltpu.VMEM((2, 2, IB, K), jnp.int32),
            pltpu.VMEM((K, d), jnp.float32),
            pltpu.VMEM((K, d), jnp.float32),
            pltpu.VMEM_SHARED((NPAD, d), jnp.float32),
            pltpu.SemaphoreType.DMA,
            pltpu.SemaphoreType.DMA,
            pltpu.SemaphoreType.DMA,
        ],
    )
    def scat_kernel(hs_hbm, e_hbm, out_hbm,
                    ib, rows0, rows1, acc_sh, sem0, sem1, semi):
        c = lax.axis_index("c")
        s = lax.axis_index("s")
        wid = s * NC + c
        rows = (rows0, rows1)
        sems = (sem0, sem1)

        def zfill(r, _):
            for i in range(d // 16):
                rows0[r, pl.ds(i * 16, 16)] = jnp.zeros((16,), jnp.float32)
            return 0

        lax.fori_loop(0, K, zfill, 0)
        for i in range(RPT // K):
            pltpu.sync_copy(rows0, acc_sh.at[pl.ds(s * RPT + i * K, K)])
        pltpu.sync_copy(e_hbm.at[wid, 0], ib.at[0])
        plsc.subcore_barrier()

        def body(b, _):
            p = lax.rem(b, 2)

            @pl.when(b < NB - 1)
            def _():
                pltpu.async_copy(e_hbm.at[wid, b + 1], ib.at[1 - p], semi)

            pltpu.async_copy(hs_hbm.at[ib.at[p, 0, 0]], rows0, sem0)
            for ch in range(IB):
                pltpu.make_async_copy(hs_hbm.at[ib.at[p, 0, ch]],
                                      rows[ch % 2], sems[ch % 2]).wait()
                if ch + 1 < IB:
                    pltpu.async_copy(hs_hbm.at[ib.at[p, 0, ch + 1]],
                                     rows[(ch + 1) % 2], sems[(ch + 1) % 2])
                pltpu.sync_copy(rows[ch % 2], acc_sh.at[ib.at[p, 1, ch]],
                                add=True)

            @pl.when(b < NB - 1)
            def _():
                pltpu.make_async_copy(e_hbm.at[wid, b + 1], ib.at[1 - p],
                                      semi).wait()

            return 0

        lax.fori_loop(0, NB, body, 0)
        plsc.subcore_barrier()
        pltpu.sync_copy(acc_sh.at[pl.ds(s * RPT, RPT)],
                        out_hbm.at[c, pl.ds(s * RPT, RPT)])

    return scat_kernel(hs, e_r)


def _tc1_body(x_ref, w_ref, degt_ref, hs_ref, dinv_ref):
    deg = degt_ref[:N_NODES, 0:1] + degt_ref[:N_NODES, 1:2] + 1.0
    dinv = lax.rsqrt(deg)
    h = jnp.dot(x_ref[...], w_ref[...], preferred_element_type=jnp.float32)
    hs_ref[...] = dinv * h
    dinv_ref[...] = dinv


def _tc2_body(p_ref, hs1_ref, dinv_ref, b1_ref, w2_ref, hs2_ref):
    dinv = dinv_ref[...]
    z = dinv * (p_ref[0, :N_NODES, :] + p_ref[1, :N_NODES, :] + hs1_ref[...])
    z = z + b1_ref[...]
    a = _SELU_SCALE * jnp.where(z > 0, z, _SELU_ALPHA * (jnp.exp(z) - 1.0))
    h2 = jnp.dot(a, w2_ref[...], preferred_element_type=jnp.float32)
    hs2_ref[...] = dinv * h2


def _tc3_body(q_ref, hs2_ref, dinv_ref, b2_ref, out_ref):
    z = dinv_ref[...] * (q_ref[0, :N_NODES, :] + q_ref[1, :N_NODES, :]
                         + hs2_ref[...])
    out_ref[...] = z + b2_ref[...]


def kernel(x, edge_index, W1, b1, W2, b2):
    ei = edge_index.astype(jnp.int32)
    npadding = E_PAD - N_EDGES
    pad_src = (jnp.arange(npadding, dtype=jnp.int32) * 13) % N_NODES
    pad_dst = N_NODES + (jnp.arange(npadding, dtype=jnp.int32)
                         % (NPAD - N_NODES))
    src_full = jnp.concatenate([ei[0], pad_src])
    dst_full = jnp.concatenate([ei[1], pad_dst])
    dst_r = dst_full.reshape(NW, CH, K)
    e_r = jnp.stack([src_full.reshape(NW, NB, IB, K),
                     dst_full.reshape(NW, NB, IB, K)], axis=2)

    deg_p = _sc_degree(dst_r)
    degt = deg_p.T

    hs1, dinv = pl.pallas_call(
        _tc1_body,
        out_shape=(jax.ShapeDtypeStruct((N_NODES, HID_DIM), jnp.float32),
                   jax.ShapeDtypeStruct((N_NODES, 1), jnp.float32)),
    )(x, W1, degt)

    p = _sc_scatter(hs1, e_r, HID_DIM)

    hs2 = pl.pallas_call(
        _tc2_body,
        out_shape=jax.ShapeDtypeStruct((N_NODES, OUT_DIM), jnp.float32),
    )(p, hs1, dinv, b1.reshape(1, HID_DIM), W2)

    q = _sc_scatter(hs2, e_r, OUT_DIM, tc_tiling=False)

    out = pl.pallas_call(
        _tc3_body,
        out_shape=jax.ShapeDtypeStruct((N_NODES, OUT_DIM), jnp.float32),
    )(q, hs2, dinv, b2.reshape(1, OUT_DIM))
    return out


# speedup vs baseline: 26.7611x; 1.1306x over previous
import functools

import jax
import jax.numpy as jnp
from jax import lax
from jax.experimental import pallas as pl
from jax.experimental.pallas import tpu as pltpu
from jax.experimental.pallas import tpu_sc as plsc

N_NODES = 10000
NPAD = 10240
IN_DIM = 128
HID_DIM = 128
OUT_DIM = 64
N_EDGES = 320000

NC = 2
NS = 16
NW = NC * NS
EPW = 10240
E_PAD = NW * EPW
K = 80
CH = EPW // K
IB = 8
NB = CH // IB
RPT = NPAD // NS

_SELU_ALPHA = 1.6732632423543772
_SELU_SCALE = 1.0507009873554805


def _mesh():
    return plsc.VectorSubcoreMesh(core_axis_name="c", subcore_axis_name="s")


def _sc_degree(dst_r):
    @functools.partial(
        pl.kernel,
        out_type=jax.ShapeDtypeStruct((NC, NPAD), jnp.float32),
        mesh=_mesh(),
        scratch_types=[
            pltpu.VMEM((CH, K), jnp.int32),
            pltpu.VMEM((K,), jnp.float32),
            pltpu.VMEM((RPT,), jnp.float32),
            pltpu.VMEM_SHARED((NPAD,), jnp.float32),
        ],
    )
    def deg_kernel(dst_hbm, out_hbm, dstv, onesv, zv, acc_sh):
        c = lax.axis_index("c")
        s = lax.axis_index("s")
        wid = s * NC + c

        def fill(i, _):
            zv[pl.ds(i * 16, 16)] = jnp.zeros((16,), jnp.float32)
            return 0

        lax.fori_loop(0, RPT // 16, fill, 0)
        for i in range(K // 16):
            onesv[pl.ds(i * 16, 16)] = jnp.ones((16,), jnp.float32)
        pltpu.sync_copy(zv, acc_sh.at[pl.ds(s * RPT, RPT)])
        pltpu.sync_copy(dst_hbm.at[wid], dstv)
        plsc.subcore_barrier()

        def body(j, _):
            pltpu.sync_copy(onesv, acc_sh.at[dstv.at[j]], add=True)
            return 0

        lax.fori_loop(0, CH, body, 0)
        plsc.subcore_barrier()
        pltpu.sync_copy(acc_sh.at[pl.ds(s * RPT, RPT)],
                        out_hbm.at[c, pl.ds(s * RPT, RPT)])

    return deg_kernel(dst_r)


def _sc_scatter(hs, src_r, dst_r, d, tc_tiling=True):
    @functools.partial(
        pl.kernel,
        out_type=jax.ShapeDtypeStruct((NC, NPAD, d), jnp.float32),
        mesh=_mesh(),
        compiler_params=pltpu.CompilerParams(use_tc_tiling_on_sc=tc_tiling),
        scratch_types=[
            pltpu.VMEM((2, IB, K), jnp.int32),
            pltpu.VMEM((2, IB, K), jnp.int32),
            pltpu.VMEM((K, d), jnp.float32),
            pltpu.VMEM((K, d), jnp.float32),
            pltpu.VMEM_SHARED((NPAD, d), jnp.float32),
            pltpu.SemaphoreType.DMA,
            pltpu.SemaphoreType.DMA,
            pltpu.SemaphoreType.DMA,
        ],
    )
    def scat_kernel(hs_hbm, src_hbm, dst_hbm, out_hbm,
                    sib, dib, rows0, rows1, acc_sh, sem0, sem1, semi):
        c = lax.axis_index("c")
        s = lax.axis_index("s")
        wid = s * NC + c
        rows = (rows0, rows1)
        sems = (sem0, sem1)

        def zfill(r, _):
            for i in range(d // 16):
                rows0[r, pl.ds(i * 16, 16)] = jnp.zeros((16,), jnp.float32)
            return 0

        lax.fori_loop(0, K, zfill, 0)
        for i in range(RPT // K):
            pltpu.sync_copy(rows0, acc_sh.at[pl.ds(s * RPT + i * K, K)])
        pltpu.sync_copy(src_hbm.at[wid, 0], sib.at[0])
        pltpu.sync_copy(dst_hbm.at[wid, 0], dib.at[0])
        plsc.subcore_barrier()
        pltpu.async_copy(hs_hbm.at[sib.at[0, 0]], rows0, sem0)

        def body(b, _):
            p = lax.rem(b, 2)

            @pl.when(b < NB - 1)
            def _():
                pltpu.async_copy(src_hbm.at[wid, b + 1], sib.at[1 - p], semi)
                pltpu.async_copy(dst_hbm.at[wid, b + 1], dib.at[1 - p], semi)

            for ch in range(IB):
                pltpu.make_async_copy(hs_hbm.at[sib.at[p, ch]],
                                      rows[ch % 2], sems[ch % 2]).wait()
                if ch + 1 < IB:
                    pltpu.async_copy(hs_hbm.at[sib.at[p, ch + 1]],
                                     rows[(ch + 1) % 2], sems[(ch + 1) % 2])
                else:
                    @pl.when(b < NB - 1)
                    def _():
                        pltpu.make_async_copy(src_hbm.at[wid, b + 1],
                                              sib.at[1 - p], semi).wait()
                        pltpu.make_async_copy(dst_hbm.at[wid, b + 1],
                                              dib.at[1 - p], semi).wait()
                        pltpu.async_copy(hs_hbm.at[sib.at[1 - p, 0]],
                                         rows0, sem0)
                pltpu.sync_copy(rows[ch % 2], acc_sh.at[dib.at[p, ch]],
                                add=True)
            return 0

        lax.fori_loop(0, NB, body, 0)
        plsc.subcore_barrier()
        pltpu.sync_copy(acc_sh.at[pl.ds(s * RPT, RPT)],
                        out_hbm.at[c, pl.ds(s * RPT, RPT)])

    return scat_kernel(hs, src_r, dst_r)


def _tc1_body(x_ref, w_ref, degt_ref, hs_ref, dinv_ref):
    deg = degt_ref[:N_NODES, 0:1] + degt_ref[:N_NODES, 1:2] + 1.0
    dinv = lax.rsqrt(deg)
    h = jnp.dot(x_ref[...], w_ref[...], preferred_element_type=jnp.float32)
    hs_ref[...] = dinv * h
    dinv_ref[...] = dinv


def _tc2_body(p_ref, hs1_ref, dinv_ref, b1_ref, w2_ref, hs2_ref):
    dinv = dinv_ref[...]
    z = dinv * (p_ref[0, :N_NODES, :] + p_ref[1, :N_NODES, :] + hs1_ref[...])
    z = z + b1_ref[...]
    a = _SELU_SCALE * jnp.where(z > 0, z, _SELU_ALPHA * (jnp.exp(z) - 1.0))
    h2 = jnp.dot(a, w2_ref[...], preferred_element_type=jnp.float32)
    hs2_ref[...] = dinv * h2


def _tc3_body(q_ref, hs2_ref, dinv_ref, b2_ref, out_ref):
    z = dinv_ref[...] * (q_ref[0, :N_NODES, :] + q_ref[1, :N_NODES, :]
                         + hs2_ref[...])
    out_ref[...] = z + b2_ref[...]


def kernel(x, edge_index, W1, b1, W2, b2):
    ei = edge_index.astype(jnp.int32)
    npadding = E_PAD - N_EDGES
    pad_src = (jnp.arange(npadding, dtype=jnp.int32) * 13) % N_NODES
    pad_dst = N_NODES + (jnp.arange(npadding, dtype=jnp.int32)
                         % (NPAD - N_NODES))
    src_full = jnp.concatenate([ei[0], pad_src])
    dst_full = jnp.concatenate([ei[1], pad_dst])
    dst_r = dst_full.reshape(NW, CH, K)
    src_rb = src_full.reshape(NW, NB, IB, K)
    dst_rb = dst_full.reshape(NW, NB, IB, K)

    deg_p = _sc_degree(dst_r)
    degt = deg_p.T

    hs1, dinv = pl.pallas_call(
        _tc1_body,
        out_shape=(jax.ShapeDtypeStruct((N_NODES, HID_DIM), jnp.float32),
                   jax.ShapeDtypeStruct((N_NODES, 1), jnp.float32)),
    )(x, W1, degt)

    p = _sc_scatter(hs1, src_rb, dst_rb, HID_DIM)

    hs2 = pl.pallas_call(
        _tc2_body,
        out_shape=jax.ShapeDtypeStruct((N_NODES, OUT_DIM), jnp.float32),
    )(p, hs1, dinv, b1.reshape(1, HID_DIM), W2)

    q = _sc_scatter(hs2, src_rb, dst_rb, OUT_DIM, tc_tiling=False)

    out = pl.pallas_call(
        _tc3_body,
        out_shape=jax.ShapeDtypeStruct((N_NODES, OUT_DIM), jnp.float32),
    )(q, hs2, dinv, b2.reshape(1, OUT_DIM))
    return out
